# baseline (speedup 1.0000x reference)
import functools

import jax
import jax.numpy as jnp
from jax.experimental import pallas as pl
from jax.experimental.pallas import tpu as pltpu


def _drqn_step(x_ref, h0_ref, w0_ref, wih2_ref, whh1_ref, whh2_ref,
               b1_ref, b2_ref, wq_ref, bq_ref,
               out_ref, hout_ref, h1_ref, h2_ref, *, T):
    c = pl.program_id(0)
    B = h0_ref.shape[1]
    f32 = jnp.float32
    bf16 = jnp.bfloat16

    @pl.when(c == 0)
    def init_state():
        h1_ref[...] = h0_ref[0].astype(bf16)
        h2_ref[...] = h0_ref[1].astype(bf16)

    h1_prev = h1_ref[...]
    h2_prev = h2_ref[...]

    @pl.when(c < T)
    def layer1():
        x = x_ref[...].reshape(B, -1).astype(bf16)
        pre = jnp.dot(x, w0_ref[...], preferred_element_type=f32) + b1_ref[...]
        rec = jnp.dot(h1_prev, whh1_ref[...], preferred_element_type=f32)
        h1 = jnp.tanh(pre + rec)
        h1_ref[...] = h1.astype(bf16)

        @pl.when(c == T - 1)
        def _():
            hout_ref[0] = h1

    @pl.when(c > 0)
    def layer2():
        rec = jnp.dot(h2_prev, whh2_ref[...], preferred_element_type=f32)
        h2 = jnp.tanh(jnp.dot(h1_prev, wih2_ref[...],
                              preferred_element_type=f32) + b2_ref[...] + rec)
        h2b = h2.astype(bf16)
        h2_ref[...] = h2b
        q = jnp.dot(h2b, wq_ref[...], preferred_element_type=f32) + bq_ref[...]
        out_ref[...] = q.reshape(1, B, -1)

        @pl.when(c == T)
        def _():
            hout_ref[1] = h2


def kernel(inputs, hidden_state, w_ih0, w_ih, w_hh, b_ih, b_hh, w_q, b_q):
    B, T, D = inputs.shape
    L, _, H = hidden_state.shape
    R = w_q.shape[0]

    x4 = inputs.reshape(B, T, 1, D)

    w0_t = w_ih0.T.astype(jnp.bfloat16)
    wih2_t = w_ih[1].T.astype(jnp.bfloat16)
    whh1_t = w_hh[0].T.astype(jnp.bfloat16)
    whh2_t = w_hh[1].T.astype(jnp.bfloat16)
    bias = b_ih + b_hh
    b1 = bias[0].reshape(1, H)
    b2 = bias[1].reshape(1, H)
    wq_t = w_q.T.astype(jnp.bfloat16)
    bq = b_q.reshape(1, R)

    full = lambda shape: pl.BlockSpec(shape, lambda c: (0,) * len(shape))

    out_tm, h_out = pl.pallas_call(
        functools.partial(_drqn_step, T=T),
        grid=(T + 1,),
        in_specs=[
            pl.BlockSpec((B, 1, 1, D),
                         lambda c: (0, jnp.minimum(c, T - 1), 0, 0)),
            full((L, B, H)),
            full((D, H)),
            full((H, H)),
            full((H, H)),
            full((H, H)),
            full((1, H)),
            full((1, H)),
            full((H, R)),
            full((1, R)),
        ],
        out_specs=(
            pl.BlockSpec((1, B, R),
                         lambda c: (jnp.clip(c - 1, 0, T - 1), 0, 0)),
            full((L, B, H)),
        ),
        out_shape=(
            jax.ShapeDtypeStruct((T, B, R), jnp.float32),
            jax.ShapeDtypeStruct((L, B, H), jnp.float32),
        ),
        scratch_shapes=[
            pltpu.VMEM((B, H), jnp.bfloat16),
            pltpu.VMEM((B, H), jnp.bfloat16),
        ],
        compiler_params=pltpu.CompilerParams(
            dimension_semantics=("arbitrary",)),
    )(x4, hidden_state, w0_t, wih2_t, whh1_t, whh2_t, b1, b2, wq_t, bq)

    out = jnp.transpose(out_tm, (1, 0, 2))
    return out, h_out


# speedup vs baseline: 1.7461x; 1.7461x over previous
import jax
import jax.numpy as jnp
from jax.experimental import pallas as pl
from jax.experimental.pallas import tpu as pltpu

_SLOTS = 8
_PREFETCH = 6


def _drqn_body(x_hbm, h0_ref, w0_ref, wih2_ref, whh1_ref, whh2_ref,
               b1_ref, b2_ref, wq_ref, bq_ref,
               out_ref, hout_ref, ring_ref, sems):
    B, T, D = x_hbm.shape
    f32 = jnp.float32
    bf16 = jnp.bfloat16

    def start_fetch(t):
        pltpu.make_async_copy(
            x_hbm.at[:, t], ring_ref.at[t % _SLOTS],
            sems.at[t % _SLOTS]).start()

    def wait_fetch(t):
        slot = t % _SLOTS
        pltpu.make_async_copy(
            ring_ref.at[slot], ring_ref.at[slot], sems.at[slot]).wait()

    for t in range(min(_PREFETCH, T)):
        start_fetch(t)

    w0 = w0_ref[...]
    wih2 = wih2_ref[...]
    whh1 = whh1_ref[...]
    whh2 = whh2_ref[...]
    wq = wq_ref[...]
    b1 = b1_ref[...]
    b2 = b2_ref[...]
    bq = bq_ref[...]

    h1b = h0_ref[0].astype(bf16)
    h2b = h0_ref[1].astype(bf16)

    for r in range(T + 1):
        h1b_old = h1b
        if r < T:
            wait_fetch(r)
            xb = ring_ref[r % _SLOTS].astype(bf16)
            pre = jnp.dot(xb, w0, preferred_element_type=f32)
            m1 = jnp.dot(h1b_old, whh1, preferred_element_type=f32)
            h1 = jnp.tanh(pre + b1 + m1)
            h1b = h1.astype(bf16)
            if r + _PREFETCH < T:
                start_fetch(r + _PREFETCH)
            if r == T - 1:
                hout_ref[0] = h1
        if r > 0:
            m2i = jnp.dot(h1b_old, wih2, preferred_element_type=f32)
            m2h = jnp.dot(h2b, whh2, preferred_element_type=f32)
            h2 = jnp.tanh(m2i + b2 + m2h)
            h2b = h2.astype(bf16)
            q = jnp.dot(h2b, wq, preferred_element_type=f32) + bq
            out_ref[:, r - 1, :] = q
            if r == T:
                hout_ref[1] = h2


def kernel(inputs, hidden_state, w_ih0, w_ih, w_hh, b_ih, b_hh, w_q, b_q):
    B, T, D = inputs.shape
    L, _, H = hidden_state.shape
    R = w_q.shape[0]

    w0_t = w_ih0.T.astype(jnp.bfloat16)
    wih2_t = w_ih[1].T.astype(jnp.bfloat16)
    whh1_t = w_hh[0].T.astype(jnp.bfloat16)
    whh2_t = w_hh[1].T.astype(jnp.bfloat16)
    bias = b_ih + b_hh
    b1 = bias[0].reshape(1, H)
    b2 = bias[1].reshape(1, H)
    wq_t = w_q.T.astype(jnp.bfloat16)
    bq = b_q.reshape(1, R)

    full = lambda shape: pl.BlockSpec(shape, lambda: (0,) * len(shape))

    out, h_out = pl.pallas_call(
        _drqn_body,
        grid=(),
        in_specs=[
            pl.BlockSpec(memory_space=pl.ANY),
            full((L, B, H)),
            full((D, H)),
            full((H, H)),
            full((H, H)),
            full((H, H)),
            full((1, H)),
            full((1, H)),
            full((H, R)),
            full((1, R)),
        ],
        out_specs=(
            full((B, T, R)),
            full((L, B, H)),
        ),
        out_shape=(
            jax.ShapeDtypeStruct((B, T, R), jnp.float32),
            jax.ShapeDtypeStruct((L, B, H), jnp.float32),
        ),
        scratch_shapes=[
            pltpu.VMEM((_SLOTS, B, D), jnp.float32),
            pltpu.SemaphoreType.DMA((_SLOTS,)),
        ],
    )(inputs, hidden_state, w0_t, wih2_t, whh1_t, whh2_t, b1, b2, wq_t, bq)

    return out, h_out


# speedup vs baseline: 1.7830x; 1.0212x over previous
import jax
import jax.numpy as jnp
from jax.experimental import pallas as pl
from jax.experimental.pallas import tpu as pltpu

_SLOTS = 8
_PREFETCH = 6


def _drqn_body(x_hbm, h0_ref, w_ref, wq_ref, b_ref, bq_ref,
               out_ref, hout_ref, ring_ref, sems):
    B, T, D = x_hbm.shape
    H = h0_ref.shape[2]
    f32 = jnp.float32
    bf16 = jnp.bfloat16

    def start_fetch(t):
        pltpu.make_async_copy(
            x_hbm.at[:, t], ring_ref.at[t % _SLOTS],
            sems.at[t % _SLOTS]).start()

    def wait_fetch(t):
        slot = t % _SLOTS
        pltpu.make_async_copy(
            ring_ref.at[slot], ring_ref.at[slot], sems.at[slot]).wait()

    for t in range(min(_PREFETCH, T)):
        start_fetch(t)

    w1 = w_ref[:, :H]
    w2 = w_ref[:, H:]
    wq = wq_ref[...]
    b1 = b_ref[0]
    b2 = b_ref[1]
    bq = bq_ref[...]

    h1b = h0_ref[0].astype(bf16)
    h2b = h0_ref[1].astype(bf16)

    for r in range(T + 1):
        h1b_old = h1b
        if r < T:
            wait_fetch(r)
            xb = ring_ref[r % _SLOTS].astype(bf16)
            xh = jnp.concatenate([xb, h1b_old], axis=1)
            h1 = jnp.tanh(
                jnp.dot(xh, w1, preferred_element_type=f32) + b1)
            h1b = h1.astype(bf16)
            if r + _PREFETCH < T:
                start_fetch(r + _PREFETCH)
            if r == T - 1:
                hout_ref[0] = h1
        if r > 0:
            hh = jnp.concatenate([h1b_old, h2b], axis=1)
            h2 = jnp.tanh(
                jnp.dot(hh, w2, preferred_element_type=f32) + b2)
            h2b = h2.astype(bf16)
            q = jnp.dot(h2b, wq, preferred_element_type=f32) + bq
            out_ref[:, r - 1, :] = q
            if r == T:
                hout_ref[1] = h2


def kernel(inputs, hidden_state, w_ih0, w_ih, w_hh, b_ih, b_hh, w_q, b_q):
    B, T, D = inputs.shape
    L, _, H = hidden_state.shape
    R = w_q.shape[0]

    w1 = jnp.concatenate([w_ih0, w_hh[0]], axis=1)
    w2 = jnp.concatenate([w_ih[1], w_hh[1]], axis=1)
    w = jnp.concatenate([w1, w2], axis=0).T.astype(jnp.bfloat16)
    wq_t = w_q.T.astype(jnp.bfloat16)
    bias = (b_ih + b_hh).reshape(L, 1, H)
    bq = b_q.reshape(1, R)

    full = lambda shape: pl.BlockSpec(shape, lambda: (0,) * len(shape))

    out, h_out = pl.pallas_call(
        _drqn_body,
        grid=(),
        in_specs=[
            pl.BlockSpec(memory_space=pl.ANY),
            full((L, B, H)),
            full((D + H, 2 * H)),
            full((H, R)),
            full((L, 1, H)),
            full((1, R)),
        ],
        out_specs=(
            full((B, T, R)),
            full((L, B, H)),
        ),
        out_shape=(
            jax.ShapeDtypeStruct((B, T, R), jnp.float32),
            jax.ShapeDtypeStruct((L, B, H), jnp.float32),
        ),
        scratch_shapes=[
            pltpu.VMEM((_SLOTS, B, D), jnp.float32),
            pltpu.SemaphoreType.DMA((_SLOTS,)),
        ],
    )(inputs, hidden_state, w, wq_t, bias, bq)

    return out, h_out
